# baseline (speedup 1.0000x reference)
import jax
import jax.numpy as jnp
from jax.experimental import pallas as pl
from jax.experimental.pallas import tpu as pltpu

_C = 1024
_D = 256
_MOM = 0.1
_EPS = 1e-6


def _body(x_ref, m_ref, out_ref, acc_ref, msb_ref, colc_ref):
    i = pl.program_id(0)
    nblk = pl.num_programs(0)

    @pl.when(i == 0)
    def _():
        acc_ref[...] = jnp.zeros_like(acc_ref)
        m = m_ref[...]
        msb_ref[...] = (m * (-2.0)).astype(jnp.bfloat16)
        ones_row = jnp.ones((1, _D), dtype=jnp.float32)
        colc_ref[...] = jax.lax.dot_general(
            ones_row, m * m - (2.0 * _EPS) * m, (((1,), (1,)), ((), ())),
            preferred_element_type=jnp.float32)

    x = x_ref[...]
    br = x.shape[0]

    scores = jax.lax.dot_general(
        x.astype(jnp.bfloat16), msb_ref[...], (((1,), (1,)), ((), ())),
        preferred_element_type=jnp.float32)

    t = scores + colc_ref[...]
    minval = jnp.min(t, axis=1, keepdims=True)
    onehot = jnp.where(t == minval, 1.0, 0.0).astype(jnp.bfloat16)

    sx = jnp.sum(x, axis=1, keepdims=True)
    lane_io = jax.lax.broadcasted_iota(jnp.int32, (br, 2), 1)
    rhs = jnp.where(lane_io == 0, 1.0, sx).astype(jnp.bfloat16)
    acc_ref[...] += jax.lax.dot_general(
        onehot, rhs, (((0,), (0,)), ((), ())),
        preferred_element_type=jnp.float32)

    @pl.when(i == nblk - 1)
    def _():
        counts = acc_ref[:, 0:1]
        sums = acc_ref[:, 1:2]
        denom = jnp.maximum(counts * float(_D), 1.0)
        mean_scalar = sums / denom
        rm = m_ref[...]
        upd = _MOM * mean_scalar + (1.0 - _MOM) * rm
        out_ref[...] = jnp.where(counts > 32.0, upd, rm)


def kernel(input, running_mean):
    n, d = input.shape
    br = 2048
    grid = n // br
    new_rm = pl.pallas_call(
        _body,
        grid=(grid,),
        in_specs=[
            pl.BlockSpec((br, d), lambda i: (i, 0)),
            pl.BlockSpec((_C, d), lambda i: (0, 0)),
        ],
        out_specs=pl.BlockSpec((_C, d), lambda i: (0, 0)),
        out_shape=jax.ShapeDtypeStruct((_C, d), jnp.float32),
        scratch_shapes=[
            pltpu.VMEM((_C, 2), jnp.float32),
            pltpu.VMEM((_C, _D), jnp.bfloat16),
            pltpu.VMEM((1, _C), jnp.float32),
        ],
    )(input, running_mean)
    return input, new_rm


# speedup vs baseline: 1.0401x; 1.0401x over previous
import jax
import jax.numpy as jnp
from jax.experimental import pallas as pl
from jax.experimental.pallas import tpu as pltpu

_C = 1024
_D = 256
_MOM = 0.1
_EPS = 1e-6


def _body(x_ref, m_ref, out_ref, acc_ref, msb_ref, colc_ref):
    i = pl.program_id(0)
    nblk = pl.num_programs(0)

    @pl.when(i == 0)
    def _():
        acc_ref[...] = jnp.zeros_like(acc_ref)
        m = m_ref[...]
        msb_ref[...] = (m * (-2.0)).astype(jnp.bfloat16)
        ones_row = jnp.ones((1, _D), dtype=jnp.float32)
        colc_ref[...] = jax.lax.dot_general(
            ones_row, m * m - (2.0 * _EPS) * m, (((1,), (1,)), ((), ())),
            preferred_element_type=jnp.float32)

    x = x_ref[...]
    br = x.shape[0]

    scores = jax.lax.dot_general(
        x.astype(jnp.bfloat16), msb_ref[...], (((1,), (1,)), ((), ())),
        preferred_element_type=jnp.float32)

    t = scores + colc_ref[...]
    minval = jnp.min(t, axis=1, keepdims=True)
    onehot = jnp.where(t == minval, 1.0, 0.0).astype(jnp.bfloat16)

    sx = jnp.sum(x, axis=1, keepdims=True)
    lane_io = jax.lax.broadcasted_iota(jnp.int32, (br, 2), 1)
    rhs = jnp.where(lane_io == 0, 1.0, sx).astype(jnp.bfloat16)
    acc_ref[...] += jax.lax.dot_general(
        onehot, rhs, (((0,), (0,)), ((), ())),
        preferred_element_type=jnp.float32)

    @pl.when(i == nblk - 1)
    def _():
        counts = acc_ref[:, 0:1]
        sums = acc_ref[:, 1:2]
        denom = jnp.maximum(counts * float(_D), 1.0)
        mean_scalar = sums / denom
        rm = m_ref[...]
        upd = _MOM * mean_scalar + (1.0 - _MOM) * rm
        out_ref[...] = jnp.where(counts > 32.0, upd, rm)


def kernel(input, running_mean):
    n, d = input.shape
    br = 4096
    grid = n // br
    new_rm = pl.pallas_call(
        _body,
        grid=(grid,),
        in_specs=[
            pl.BlockSpec((br, d), lambda i: (i, 0)),
            pl.BlockSpec((_C, d), lambda i: (0, 0)),
        ],
        out_specs=pl.BlockSpec((_C, d), lambda i: (0, 0)),
        out_shape=jax.ShapeDtypeStruct((_C, d), jnp.float32),
        scratch_shapes=[
            pltpu.VMEM((_C, 2), jnp.float32),
            pltpu.VMEM((_C, _D), jnp.bfloat16),
            pltpu.VMEM((1, _C), jnp.float32),
        ],
    )(input, running_mean)
    return input, new_rm
